# baseline (speedup 1.0000x reference)
import functools

import jax
import jax.numpy as jnp
from jax import lax
from jax.experimental import pallas as pl
from jax.experimental.pallas import tpu as pltpu
from jax.experimental.pallas import tpu_sc as plsc

_V = 100000
_D = 128
_B = 4096
_NS = 20
_NTILE = 16
_NW = 32
_VP = 102400
_CHUNK = _VP // _NTILE
_NVREG = _CHUNK // 16
_BG = _B // _NW
_CT = 1024
_GRID = _B // _CT

_SC_PARAMS = pltpu.CompilerParams(needs_layout_passes=False)


def _mesh():
    return plsc.VectorSubcoreMesh(core_axis_name="c", subcore_axis_name="s",
                                  num_cores=2, num_subcores=_NTILE)


def _merge16(ak, av, bk, bv):
    brk = lax.rev(bk, (0,))
    brv = lax.rev(bv, (0,))
    m = ak >= brk
    hik = jnp.where(m, ak, brk)
    hiv = jnp.where(m, av, brv)
    lok = jnp.where(m, brk, ak)
    lov = jnp.where(m, brv, av)
    hik, hiv = plsc.sort_key_val(hik, hiv)
    lok, lov = plsc.sort_key_val(lok, lov)
    return hik, hiv, lok, lov


def _gather_body(tgt_hbm, w_hbm, b128_hbm, wt_out, bt_out,
                 rows_v, brows_v, bstage_v, gidx_v, ridx_v, sem1, sem2):
    cid = lax.axis_index("c")
    sid = lax.axis_index("s")
    lane = lax.iota(jnp.int32, 16)
    wid = sid * 2 + cid
    base = wid * _BG
    pltpu.sync_copy(tgt_hbm.at[pl.ds(base, _BG)], gidx_v)
    wcopy = pltpu.async_copy(w_hbm.at[gidx_v], rows_v, sem1)

    def rstep(j, carry):
        r = j * 16 + lane
        tv = plsc.load_gather(gidx_v, [r])
        plsc.store_scatter(ridx_v, [r], tv >> 7)
        return carry

    lax.fori_loop(0, _BG // 16, rstep, 0)
    bcopy = pltpu.async_copy(b128_hbm.at[ridx_v], brows_v, sem2)
    wcopy.wait()
    pltpu.sync_copy(rows_v, wt_out.at[pl.ds(base, _BG)])
    bcopy.wait()

    def bstep(j, carry):
        r = j * 16 + lane
        tv = plsc.load_gather(gidx_v, [r])
        val = plsc.load_gather(brows_v, [r, tv & 127])
        plsc.store_scatter(bstage_v, [r], val)
        return carry

    lax.fori_loop(0, _BG // 16, bstep, 0)
    pltpu.sync_copy(bstage_v, bt_out.at[pl.ds(base, _BG)])


@functools.cache
def _make_gather_kernel():
    return functools.partial(
        pl.kernel,
        out_type=[
            jax.ShapeDtypeStruct((_B, _D), jnp.float32),
            jax.ShapeDtypeStruct((_B,), jnp.float32),
        ],
        mesh=_mesh(),
        scratch_types=[
            pltpu.VMEM((_BG, _D), jnp.float32),
            pltpu.VMEM((_BG, _D), jnp.float32),
            pltpu.VMEM((_BG,), jnp.float32),
            pltpu.VMEM((_BG,), jnp.int32),
            pltpu.VMEM((_BG,), jnp.int32),
            pltpu.SemaphoreType.DMA,
            pltpu.SemaphoreType.DMA,
        ],
        compiler_params=_SC_PARAMS,
    )(_gather_body)


def _sample_body(lp_hbm, e0_hbm, tgt_hbm, w_hbm, b128_hbm,
                 wn_out, bn_out,
                 lp_v, e0_v, tgt_v, rows_v, bstage_v, nidx_v, nridx_v,
                 ck_v, ci_v, t1k, t1v, t2k, t2v, thr_s,
                 candk_sh, candi_sh, sem1):
    cid = lax.axis_index("c")
    sid = lax.axis_index("s")
    lane = lax.iota(jnp.int32, 16)

    @pl.when(cid == 0)
    def _sample():
        base = sid * _CHUNK
        pltpu.sync_copy(lp_hbm.at[pl.ds(base, _CHUNK)], lp_v)
        pltpu.sync_copy(e0_hbm.at[pl.ds(base, _CHUNK)], e0_v)
        pltpu.sync_copy(tgt_hbm, tgt_v)

        zero16 = jnp.zeros((16,), jnp.float32)

        def zstep(j, carry):
            for u in range(4):
                tv = tgt_v[pl.ds((j * 4 + u) * 16, 16)]
                msk = (tv >= base) & (tv < base + _CHUNK)
                loc = jnp.where(msk, tv - base, 0)
                plsc.store_scatter(lp_v, [loc], zero16, mask=msk)
            return carry

        lax.fori_loop(0, _B // 64, zstep, 0)

        neg1 = jnp.full((16,), -1.0, jnp.float32)
        zi = jnp.zeros((16,), jnp.int32)
        t1k[...] = neg1
        t1v[...] = zi
        t2k[...] = neg1
        t2v[...] = zi
        thr_s[0] = -1.0

        def sstep(i, carry):
            off = i * 128
            idxs, dks = [], []
            for u in range(8):
                idx = off + u * 16 + lane
                dk = lp_v[pl.ds(off + u * 16, 16)] * e0_v[pl.ds(off + u * 16, 16)]
                idxs.append(idx)
                dks.append(dk)
            m01 = jnp.maximum(dks[0], dks[1])
            m23 = jnp.maximum(dks[2], dks[3])
            m45 = jnp.maximum(dks[4], dks[5])
            m67 = jnp.maximum(dks[6], dks[7])
            gmax = jnp.maximum(jnp.maximum(m01, m23), jnp.maximum(m45, m67))

            @pl.when(jnp.max(gmax) > thr_s[0])
            def _m():
                for u in range(8):
                    @pl.when(jnp.max(dks[u]) > thr_s[0])
                    def _mu(u=u):
                        dks_s, dvs_s = plsc.sort_key_val(dks[u], idxs[u] + base)
                        h1k, h1v, lok, lov = _merge16(t1k[...], t1v[...], dks_s, dvs_s)
                        h2k, h2v, _, _ = _merge16(t2k[...], t2v[...], lok, lov)
                        t1k[...] = h1k
                        t1v[...] = h1v
                        t2k[...] = h2k
                        t2v[...] = h2v
                        thr_s[0] = jnp.min(h2k)

            return carry

        lax.fori_loop(0, _NVREG // 8, sstep, 0)

        ck_v[pl.ds(0, 16)] = t1k[...]
        ck_v[pl.ds(16, 16)] = t2k[...]
        ci_v[pl.ds(0, 16)] = t1v[...]
        ci_v[pl.ds(16, 16)] = t2v[...]
        pltpu.sync_copy(ck_v, candk_sh.at[pl.ds(sid * 32, 32)])
        pltpu.sync_copy(ci_v, candi_sh.at[pl.ds(sid * 32, 32)])
        plsc.subcore_barrier()

        @pl.when(sid == 0)
        def _final():
            pltpu.sync_copy(candk_sh, lp_v.at[pl.ds(0, 512)])
            pltpu.sync_copy(candi_sh, tgt_v.at[pl.ds(0, 512)])
            t1k[...] = neg1
            t1v[...] = zi
            t2k[...] = neg1
            t2v[...] = zi
            for i in range(32):
                dk = lp_v[pl.ds(i * 16, 16)]
                dv = tgt_v[pl.ds(i * 16, 16)]
                h1k, h1v, lok, lov = _merge16(t1k[...], t1v[...], dk, dv)
                h2k, h2v, _, _ = _merge16(t2k[...], t2v[...], lok, lov)
                t1k[...] = h1k
                t1v[...] = h1v
                t2k[...] = h2k
                t2v[...] = h2v
            for i in range(8):
                nidx_v[pl.ds(i * 16, 16)] = zi
            nidx_v[pl.ds(0, 16)] = t1v[...]
            plsc.store_scatter(nidx_v, [lane + 4], t2v[...], mask=lane >= 12)
            pltpu.async_copy(w_hbm.at[nidx_v], rows_v, sem1).wait()
            pltpu.sync_copy(rows_v, wn_out)
            for j in range(8):
                r = j * 16 + lane
                nv = plsc.load_gather(nidx_v, [r])
                plsc.store_scatter(nridx_v, [r], nv >> 7)
            pltpu.async_copy(b128_hbm.at[nridx_v], rows_v, sem1).wait()
            for j in range(8):
                r = j * 16 + lane
                nv = plsc.load_gather(nidx_v, [r])
                val = plsc.load_gather(rows_v, [r, nv & 127])
                plsc.store_scatter(bstage_v, [r], val)
            pltpu.sync_copy(bstage_v, bn_out)


@functools.cache
def _make_sample_kernel():
    return functools.partial(
        pl.kernel,
        out_type=[
            jax.ShapeDtypeStruct((128, _D), jnp.float32),
            jax.ShapeDtypeStruct((128,), jnp.float32),
        ],
        mesh=_mesh(),
        scratch_types=[
            pltpu.VMEM((_CHUNK,), jnp.float32),
            pltpu.VMEM((_CHUNK,), jnp.float32),
            pltpu.VMEM((_B,), jnp.int32),
            pltpu.VMEM((128, _D), jnp.float32),
            pltpu.VMEM((128,), jnp.float32),
            pltpu.VMEM((128,), jnp.int32),
            pltpu.VMEM((128,), jnp.int32),
            pltpu.VMEM((32,), jnp.float32),
            pltpu.VMEM((32,), jnp.int32),
            pltpu.VMEM((16,), jnp.float32),
            pltpu.VMEM((16,), jnp.int32),
            pltpu.VMEM((16,), jnp.float32),
            pltpu.VMEM((16,), jnp.int32),
            pltpu.SMEM((1,), jnp.float32),
            pltpu.VMEM_SHARED((512,), jnp.float32),
            pltpu.VMEM_SHARED((512,), jnp.int32),
            pltpu.SemaphoreType.DMA,
        ],
        compiler_params=_SC_PARAMS,
    )(_sample_body)


def _log_sigmoid(x):
    return jnp.minimum(x, 0.0) - jnp.log(1.0 + jnp.exp(-jnp.abs(x)))


def _tgt_body(f_ref, wt_ref, bt_ref, out_ref):
    j = pl.program_id(0)
    lt = lax.dot_general(f_ref[...], wt_ref[...],
                         (((1,), (1,)), ((), ())),
                         preferred_element_type=jnp.float32) + bt_ref[0]
    s = jnp.sum(_log_sigmoid(lt))

    @pl.when(j == 0)
    def _init():
        out_ref[0, 0] = s

    @pl.when(j > 0)
    def _acc():
        out_ref[0, 0] += s


_tgt_kernel = pl.pallas_call(
    _tgt_body,
    grid=(_GRID,),
    in_specs=[
        pl.BlockSpec((_B, _D), lambda j: (0, 0)),
        pl.BlockSpec((_CT, _D), lambda j: (j, 0)),
        pl.BlockSpec((1, 1, _CT), lambda j: (j, 0, 0)),
    ],
    out_specs=pl.BlockSpec(memory_space=pltpu.SMEM),
    out_shape=jax.ShapeDtypeStruct((1, 1), jnp.float32),
    compiler_params=pltpu.CompilerParams(
        dimension_semantics=("arbitrary",),
    ),
)


def _noise_body(f_ref, wn_ref, bn_ref, part_ref, out_ref):
    zn = -(lax.dot_general(f_ref[...], wn_ref[...],
                           (((1,), (1,)), ((), ())),
                           preferred_element_type=jnp.float32)
           + bn_ref[0])
    lsn = _log_sigmoid(zn)
    msk = lax.broadcasted_iota(jnp.int32, (_B, 128), 1) < _NS
    s = jnp.sum(jnp.where(msk, lsn, 0.0)) + part_ref[0, 0]
    out_ref[0, 0] = -s / (_B * (_B + _NS))


_noise_kernel = pl.pallas_call(
    _noise_body,
    in_specs=[
        pl.BlockSpec((_B, _D), lambda: (0, 0)),
        pl.BlockSpec((128, _D), lambda: (0, 0)),
        pl.BlockSpec((1, 1, 128), lambda: (0, 0, 0)),
        pl.BlockSpec(memory_space=pltpu.SMEM),
    ],
    out_specs=pl.BlockSpec(memory_space=pltpu.SMEM),
    out_shape=jax.ShapeDtypeStruct((1, 1), jnp.float32),
)


def kernel(feature, targets, W, b, label_probs):
    e0 = jnp.exp(jax.random.gumbel(jax.random.key(42), (_V,), jnp.float32))
    lp_p = jnp.pad(label_probs.astype(jnp.float32), (0, _VP - _V))
    e0_p = jnp.pad(e0, (0, _VP - _V))
    b128 = jnp.pad(b.astype(jnp.float32), (0, _VP - _V)).reshape(_VP // 128, 128)
    tgt = targets.astype(jnp.int32)

    wt, bt = _make_gather_kernel()(tgt, W, b128)
    wn, bn = _make_sample_kernel()(lp_p, e0_p, tgt, W, b128)

    part = _tgt_kernel(feature, wt, bt.reshape(_GRID, 1, _CT))
    out = _noise_kernel(feature, wn, bn.reshape(1, 1, 128), part)
    return out[0, 0]


# speedup vs baseline: 1.0878x; 1.0878x over previous
import functools

import jax
import jax.numpy as jnp
from jax import lax
from jax.experimental import pallas as pl
from jax.experimental.pallas import tpu as pltpu
from jax.experimental.pallas import tpu_sc as plsc

_V = 100000
_D = 128
_B = 4096
_NS = 20
_NTILE = 16
_NW = 32
_VP = 102400
_CHUNK = _VP // _NTILE
_NVREG = _CHUNK // 16
_BG = _B // _NW
_CT = 512
_GRID = _B // _CT


def _merge16(ak, av, bk, bv):
    brk = lax.rev(bk, (0,))
    brv = lax.rev(bv, (0,))
    m = ak >= brk
    hik = jnp.where(m, ak, brk)
    hiv = jnp.where(m, av, brv)
    lok = jnp.where(m, brk, ak)
    lov = jnp.where(m, brv, av)
    hik, hiv = plsc.sort_key_val(hik, hiv)
    lok, lov = plsc.sort_key_val(lok, lov)
    return hik, hiv, lok, lov


def _sc_body(mode_hbm, lp_hbm, e0_hbm, tgt_hbm, w_hbm, b128_hbm,
             wt_out, bt_out, wn_out, bn_out,
             lp_v, e0_v, tgt_v, rows_v, brows_v, bstage_v, gidx_v, ridx_v,
             nidx_v, nridx_v, mode_v, ck_v, ci_v, t1k, t1v, t2k, t2v, thr_s,
             candk_sh, candi_sh, sem1, sem2):
    cid = lax.axis_index("c")
    sid = lax.axis_index("s")
    lane = lax.iota(jnp.int32, 16)
    pltpu.sync_copy(mode_hbm, mode_v)
    mode = jnp.max(mode_v[...])

    @pl.when(mode == 0)
    def _gather_targets():
        wid = sid * 2 + cid
        base = wid * _BG
        pltpu.sync_copy(tgt_hbm.at[pl.ds(base, _BG)], gidx_v)
        wcopy = pltpu.async_copy(w_hbm.at[gidx_v], rows_v, sem1)

        def rstep(j, carry):
            r = j * 16 + lane
            tv = plsc.load_gather(gidx_v, [r])
            plsc.store_scatter(ridx_v, [r], tv >> 7)
            return carry

        lax.fori_loop(0, _BG // 16, rstep, 0)
        bcopy = pltpu.async_copy(b128_hbm.at[ridx_v], brows_v, sem2)
        wcopy.wait()
        pltpu.sync_copy(rows_v, wt_out.at[pl.ds(base, _BG)])
        bcopy.wait()

        def bstep(j, carry):
            r = j * 16 + lane
            tv = plsc.load_gather(gidx_v, [r])
            val = plsc.load_gather(brows_v, [r, tv & 127])
            plsc.store_scatter(bstage_v, [r], val)
            return carry

        lax.fori_loop(0, _BG // 16, bstep, 0)
        pltpu.sync_copy(bstage_v, bt_out.at[pl.ds(base, _BG)])

    @pl.when((mode == 1) & (cid == 0))
    def _sample():
        base = sid * _CHUNK
        pltpu.sync_copy(lp_hbm.at[pl.ds(base, _CHUNK)], lp_v)
        pltpu.sync_copy(e0_hbm.at[pl.ds(base, _CHUNK)], e0_v)
        pltpu.sync_copy(tgt_hbm, tgt_v)

        zero16 = jnp.zeros((16,), jnp.float32)

        def zstep(j, carry):
            for u in range(4):
                tv = plsc.load_gather(tgt_v, [(j * 4 + u) * 16 + lane])
                msk = (tv >= base) & (tv < base + _CHUNK)
                loc = jnp.where(msk, tv - base, 0)
                plsc.store_scatter(lp_v, [loc], zero16, mask=msk)
            return carry

        lax.fori_loop(0, _B // 64, zstep, 0)

        neg1 = jnp.full((16,), -1.0, jnp.float32)
        zi = jnp.zeros((16,), jnp.int32)
        t1k[...] = neg1
        t1v[...] = zi
        t2k[...] = neg1
        t2v[...] = zi
        thr_s[0] = -1.0

        def sstep(i, carry):
            idxs, dks = [], []
            for u in range(4):
                idx = (i * 4 + u) * 16 + lane
                dk = plsc.load_gather(lp_v, [idx]) * plsc.load_gather(e0_v, [idx])
                idxs.append(idx)
                dks.append(dk)
            gmax = jnp.maximum(jnp.maximum(dks[0], dks[1]),
                               jnp.maximum(dks[2], dks[3]))

            @pl.when(jnp.max(gmax) > thr_s[0])
            def _m():
                for u in range(4):
                    dks_s, dvs_s = plsc.sort_key_val(dks[u], idxs[u] + base)
                    h1k, h1v, lok, lov = _merge16(t1k[...], t1v[...], dks_s, dvs_s)
                    h2k, h2v, _, _ = _merge16(t2k[...], t2v[...], lok, lov)
                    t1k[...] = h1k
                    t1v[...] = h1v
                    t2k[...] = h2k
                    t2v[...] = h2v
                thr_s[0] = jnp.min(t2k[...])

            return carry

        lax.fori_loop(0, _NVREG // 4, sstep, 0)

        ck_v[pl.ds(0, 16)] = t1k[...]
        ck_v[pl.ds(16, 16)] = t2k[...]
        ci_v[pl.ds(0, 16)] = t1v[...]
        ci_v[pl.ds(16, 16)] = t2v[...]
        pltpu.sync_copy(ck_v, candk_sh.at[pl.ds(sid * 32, 32)])
        pltpu.sync_copy(ci_v, candi_sh.at[pl.ds(sid * 32, 32)])
        plsc.subcore_barrier()

        @pl.when(sid == 0)
        def _final():
            pltpu.sync_copy(candk_sh, lp_v.at[pl.ds(0, 512)])
            pltpu.sync_copy(candi_sh, tgt_v.at[pl.ds(0, 512)])
            t1k[...] = neg1
            t1v[...] = zi
            t2k[...] = neg1
            t2v[...] = zi

            def fstep(i, carry):
                r = i * 16 + lane
                dk = plsc.load_gather(lp_v, [r])
                dv = plsc.load_gather(tgt_v, [r])
                dks_s, dvs_s = plsc.sort_key_val(dk, dv)
                h1k, h1v, lok, lov = _merge16(t1k[...], t1v[...], dks_s, dvs_s)
                h2k, h2v, _, _ = _merge16(t2k[...], t2v[...], lok, lov)
                t1k[...] = h1k
                t1v[...] = h1v
                t2k[...] = h2k
                t2v[...] = h2v
                return carry

            lax.fori_loop(0, 32, fstep, 0)
            for i in range(8):
                nidx_v[pl.ds(i * 16, 16)] = zi
            nidx_v[pl.ds(0, 16)] = t1v[...]
            plsc.store_scatter(nidx_v, [lane + 4], t2v[...], mask=lane >= 12)
            pltpu.async_copy(w_hbm.at[nidx_v], rows_v, sem1).wait()
            pltpu.sync_copy(rows_v, wn_out)
            for j in range(8):
                r = j * 16 + lane
                nv = plsc.load_gather(nidx_v, [r])
                plsc.store_scatter(nridx_v, [r], nv >> 7)
            pltpu.async_copy(b128_hbm.at[nridx_v], brows_v, sem2).wait()
            for j in range(8):
                r = j * 16 + lane
                nv = plsc.load_gather(nidx_v, [r])
                val = plsc.load_gather(brows_v, [r, nv & 127])
                plsc.store_scatter(bstage_v, [r], val)
            pltpu.sync_copy(bstage_v, bn_out)


@functools.cache
def _make_sc_kernel():
    return functools.partial(
        pl.kernel,
        out_type=[
            jax.ShapeDtypeStruct((_B, _D), jnp.float32),
            jax.ShapeDtypeStruct((_B,), jnp.float32),
            jax.ShapeDtypeStruct((128, _D), jnp.float32),
            jax.ShapeDtypeStruct((128,), jnp.float32),
        ],
        mesh=plsc.VectorSubcoreMesh(core_axis_name="c", subcore_axis_name="s",
                                    num_cores=2, num_subcores=_NTILE),
        scratch_types=[
            pltpu.VMEM((_CHUNK,), jnp.float32),
            pltpu.VMEM((_CHUNK,), jnp.float32),
            pltpu.VMEM((_B,), jnp.int32),
            pltpu.VMEM((_BG, _D), jnp.float32),
            pltpu.VMEM((_BG, _D), jnp.float32),
            pltpu.VMEM((_BG,), jnp.float32),
            pltpu.VMEM((_BG,), jnp.int32),
            pltpu.VMEM((_BG,), jnp.int32),
            pltpu.VMEM((128,), jnp.int32),
            pltpu.VMEM((128,), jnp.int32),
            pltpu.VMEM((16,), jnp.int32),
            pltpu.VMEM((32,), jnp.float32),
            pltpu.VMEM((32,), jnp.int32),
            pltpu.VMEM((16,), jnp.float32),
            pltpu.VMEM((16,), jnp.int32),
            pltpu.VMEM((16,), jnp.float32),
            pltpu.VMEM((16,), jnp.int32),
            pltpu.SMEM((1,), jnp.float32),
            pltpu.VMEM_SHARED((512,), jnp.float32),
            pltpu.VMEM_SHARED((512,), jnp.int32),
            pltpu.SemaphoreType.DMA,
            pltpu.SemaphoreType.DMA,
        ],
        compiler_params=pltpu.CompilerParams(needs_layout_passes=False),
        name="sc_sample_gather",
    )(_sc_body)


def _log_sigmoid(x):
    return jnp.minimum(x, 0.0) - jnp.log(1.0 + jnp.exp(-jnp.abs(x)))


def _tgt_body(f_ref, wt_ref, bt_ref, out_ref):
    j = pl.program_id(0)
    lt = lax.dot_general(f_ref[...], wt_ref[...],
                         (((1,), (1,)), ((), ())),
                         preferred_element_type=jnp.float32) + bt_ref[0]
    s = jnp.sum(_log_sigmoid(lt))

    @pl.when(j == 0)
    def _init():
        out_ref[0, 0] = s

    @pl.when(j > 0)
    def _acc():
        out_ref[0, 0] += s


_tgt_kernel = pl.pallas_call(
    _tgt_body,
    grid=(_GRID,),
    in_specs=[
        pl.BlockSpec((_B, _D), lambda j: (0, 0)),
        pl.BlockSpec((_CT, _D), lambda j: (j, 0)),
        pl.BlockSpec((1, 1, _CT), lambda j: (j, 0, 0)),
    ],
    out_specs=pl.BlockSpec(memory_space=pltpu.SMEM),
    out_shape=jax.ShapeDtypeStruct((1, 1), jnp.float32),
    compiler_params=pltpu.CompilerParams(
        dimension_semantics=("arbitrary",),
    ),
)


def _noise_body(f_ref, wn_ref, bn_ref, part_ref, out_ref):
    zn = -(lax.dot_general(f_ref[...], wn_ref[...],
                           (((1,), (1,)), ((), ())),
                           preferred_element_type=jnp.float32)
           + bn_ref[0])
    lsn = _log_sigmoid(zn)
    msk = lax.broadcasted_iota(jnp.int32, (_B, 128), 1) < _NS
    s = jnp.sum(jnp.where(msk, lsn, 0.0)) + part_ref[0, 0]
    out_ref[0, 0] = -s / (_B * (_B + _NS))


_noise_kernel = pl.pallas_call(
    _noise_body,
    in_specs=[
        pl.BlockSpec((_B, _D), lambda: (0, 0)),
        pl.BlockSpec((128, _D), lambda: (0, 0)),
        pl.BlockSpec((1, 1, 128), lambda: (0, 0, 0)),
        pl.BlockSpec(memory_space=pltpu.SMEM),
    ],
    out_specs=pl.BlockSpec(memory_space=pltpu.SMEM),
    out_shape=jax.ShapeDtypeStruct((1, 1), jnp.float32),
)


def kernel(feature, targets, W, b, label_probs):
    e0 = jnp.exp(jax.random.gumbel(jax.random.key(42), (_V,), jnp.float32))
    lp_p = jnp.pad(label_probs.astype(jnp.float32), (0, _VP - _V))
    e0_p = jnp.pad(e0, (0, _VP - _V))
    b128 = jnp.pad(b.astype(jnp.float32), (0, _VP - _V)).reshape(_VP // 128, 128)
    tgt = targets.astype(jnp.int32)
    mode0 = jnp.zeros((16,), jnp.int32)
    mode1 = jnp.ones((16,), jnp.int32)

    sc = _make_sc_kernel()
    wt, bt, _, _ = sc(mode0, lp_p, e0_p, tgt, W, b128)
    _, _, wn, bn = sc(mode1, lp_p, e0_p, tgt, W, b128)

    part = _tgt_kernel(feature, wt, bt.reshape(_GRID, 1, _CT))
    out = _noise_kernel(feature, wn, bn.reshape(1, 1, 128), part)
    return out[0, 0]


# speedup vs baseline: 1.1519x; 1.0590x over previous
import functools

import jax
import jax.numpy as jnp
from jax import lax
from jax.experimental import pallas as pl
from jax.experimental.pallas import tpu as pltpu
from jax.experimental.pallas import tpu_sc as plsc

_V = 100000
_D = 128
_B = 4096
_NS = 20
_NTILE = 16
_NW = 32
_VP = 102400
_CHUNK = _VP // _NTILE
_NVREG = _CHUNK // 16
_BG = _B // _NW
_CT = 512
_GRID = _B // _CT


def _merge16(ak, av, bk, bv):
    brk = lax.rev(bk, (0,))
    brv = lax.rev(bv, (0,))
    m = ak >= brk
    hik = jnp.where(m, ak, brk)
    hiv = jnp.where(m, av, brv)
    lok = jnp.where(m, brk, ak)
    lov = jnp.where(m, brv, av)
    hik, hiv = plsc.sort_key_val(hik, hiv)
    lok, lov = plsc.sort_key_val(lok, lov)
    return hik, hiv, lok, lov


def _gather_body(tgt_hbm, w_hbm, b128_hbm, wt_out, bt_out,
                 rows_v, brows_v, bstage_v, gidx_v, ridx_v, sem1, sem2):
    cid = lax.axis_index("c")
    sid = lax.axis_index("s")
    lane = lax.iota(jnp.int32, 16)
    if True:
        wid = sid * 2 + cid
        base = wid * _BG
        pltpu.sync_copy(tgt_hbm.at[pl.ds(base, _BG)], gidx_v)
        wcopy = pltpu.async_copy(w_hbm.at[gidx_v], rows_v, sem1)

        def rstep(j, carry):
            r = j * 16 + lane
            tv = plsc.load_gather(gidx_v, [r])
            plsc.store_scatter(ridx_v, [r], tv >> 7)
            return carry

        lax.fori_loop(0, _BG // 16, rstep, 0)
        bcopy = pltpu.async_copy(b128_hbm.at[ridx_v], brows_v, sem2)
        wcopy.wait()
        pltpu.sync_copy(rows_v, wt_out.at[pl.ds(base, _BG)])
        bcopy.wait()

        def bstep(j, carry):
            r = j * 16 + lane
            tv = plsc.load_gather(gidx_v, [r])
            val = plsc.load_gather(brows_v, [r, tv & 127])
            plsc.store_scatter(bstage_v, [r], val)
            return carry

        lax.fori_loop(0, _BG // 16, bstep, 0)
        pltpu.sync_copy(bstage_v, bt_out.at[pl.ds(base, _BG)])


@functools.cache
def _make_gather_kernel():
    return functools.partial(
        pl.kernel,
        out_type=[
            jax.ShapeDtypeStruct((_B, _D), jnp.float32),
            jax.ShapeDtypeStruct((_B,), jnp.float32),
        ],
        mesh=plsc.VectorSubcoreMesh(core_axis_name="c", subcore_axis_name="s",
                                    num_cores=2, num_subcores=_NTILE),
        scratch_types=[
            pltpu.VMEM((_BG, _D), jnp.float32),
            pltpu.VMEM((_BG, _D), jnp.float32),
            pltpu.VMEM((_BG,), jnp.float32),
            pltpu.VMEM((_BG,), jnp.int32),
            pltpu.VMEM((_BG,), jnp.int32),
            pltpu.SemaphoreType.DMA,
            pltpu.SemaphoreType.DMA,
        ],
        compiler_params=pltpu.CompilerParams(needs_layout_passes=False),
        name="sc_gather",
    )(_gather_body)


def _sample_body(lp_hbm, e0_hbm, tgt_hbm, w_hbm, b128_hbm,
                 wn_out, bn_out,
                 lp_v, e0_v, tgt_v, rows_v, brows_v, bstage_v,
                 nidx_v, nridx_v, ck_v, ci_v, t1k, t1v, t2k, t2v, thr_s,
                 candk_sh, candi_sh, sem1, sem2):
    cid = lax.axis_index("c")
    sid = lax.axis_index("s")
    lane = lax.iota(jnp.int32, 16)

    @pl.when(cid == 0)
    def _sample():
        base = sid * _CHUNK
        pltpu.sync_copy(lp_hbm.at[pl.ds(base, _CHUNK)], lp_v)
        pltpu.sync_copy(e0_hbm.at[pl.ds(base, _CHUNK)], e0_v)
        pltpu.sync_copy(tgt_hbm, tgt_v)

        zero16 = jnp.zeros((16,), jnp.float32)

        def zstep(j, carry):
            for u in range(4):
                tv = plsc.load_gather(tgt_v, [(j * 4 + u) * 16 + lane])
                msk = (tv >= base) & (tv < base + _CHUNK)
                loc = jnp.where(msk, tv - base, 0)
                plsc.store_scatter(lp_v, [loc], zero16, mask=msk)
            return carry

        lax.fori_loop(0, _B // 64, zstep, 0)

        neg1 = jnp.full((16,), -1.0, jnp.float32)
        zi = jnp.zeros((16,), jnp.int32)
        t1k[...] = neg1
        t1v[...] = zi
        t2k[...] = neg1
        t2v[...] = zi
        thr_s[0] = -1.0

        def sstep(i, carry):
            idxs, dks = [], []
            for u in range(4):
                idx = (i * 4 + u) * 16 + lane
                dk = plsc.load_gather(lp_v, [idx]) * plsc.load_gather(e0_v, [idx])
                idxs.append(idx)
                dks.append(dk)
            gmax = jnp.maximum(jnp.maximum(dks[0], dks[1]),
                               jnp.maximum(dks[2], dks[3]))

            @pl.when(jnp.max(gmax) > thr_s[0])
            def _m():
                for u in range(4):
                    dks_s, dvs_s = plsc.sort_key_val(dks[u], idxs[u] + base)
                    h1k, h1v, lok, lov = _merge16(t1k[...], t1v[...], dks_s, dvs_s)
                    h2k, h2v, _, _ = _merge16(t2k[...], t2v[...], lok, lov)
                    t1k[...] = h1k
                    t1v[...] = h1v
                    t2k[...] = h2k
                    t2v[...] = h2v
                thr_s[0] = jnp.min(t2k[...])

            return carry

        lax.fori_loop(0, _NVREG // 4, sstep, 0)

        ck_v[pl.ds(0, 16)] = t1k[...]
        ck_v[pl.ds(16, 16)] = t2k[...]
        ci_v[pl.ds(0, 16)] = t1v[...]
        ci_v[pl.ds(16, 16)] = t2v[...]
        pltpu.sync_copy(ck_v, candk_sh.at[pl.ds(sid * 32, 32)])
        pltpu.sync_copy(ci_v, candi_sh.at[pl.ds(sid * 32, 32)])
        plsc.subcore_barrier()

        @pl.when(sid == 0)
        def _final():
            pltpu.sync_copy(candk_sh, lp_v.at[pl.ds(0, 512)])
            pltpu.sync_copy(candi_sh, tgt_v.at[pl.ds(0, 512)])
            t1k[...] = neg1
            t1v[...] = zi
            t2k[...] = neg1
            t2v[...] = zi

            def fstep(i, carry):
                r = i * 16 + lane
                dk = plsc.load_gather(lp_v, [r])
                dv = plsc.load_gather(tgt_v, [r])
                dks_s, dvs_s = plsc.sort_key_val(dk, dv)
                h1k, h1v, lok, lov = _merge16(t1k[...], t1v[...], dks_s, dvs_s)
                h2k, h2v, _, _ = _merge16(t2k[...], t2v[...], lok, lov)
                t1k[...] = h1k
                t1v[...] = h1v
                t2k[...] = h2k
                t2v[...] = h2v
                return carry

            lax.fori_loop(0, 32, fstep, 0)
            for i in range(8):
                nidx_v[pl.ds(i * 16, 16)] = zi
            nidx_v[pl.ds(0, 16)] = t1v[...]
            plsc.store_scatter(nidx_v, [lane + 4], t2v[...], mask=lane >= 12)
            pltpu.async_copy(w_hbm.at[nidx_v], rows_v, sem1).wait()
            pltpu.sync_copy(rows_v, wn_out)
            for j in range(8):
                r = j * 16 + lane
                nv = plsc.load_gather(nidx_v, [r])
                plsc.store_scatter(nridx_v, [r], nv >> 7)
            pltpu.async_copy(b128_hbm.at[nridx_v], brows_v, sem2).wait()
            for j in range(8):
                r = j * 16 + lane
                nv = plsc.load_gather(nidx_v, [r])
                val = plsc.load_gather(brows_v, [r, nv & 127])
                plsc.store_scatter(bstage_v, [r], val)
            pltpu.sync_copy(bstage_v, bn_out)


@functools.cache
def _make_sample_kernel():
    return functools.partial(
        pl.kernel,
        out_type=[
            jax.ShapeDtypeStruct((128, _D), jnp.float32),
            jax.ShapeDtypeStruct((128,), jnp.float32),
        ],
        mesh=plsc.VectorSubcoreMesh(core_axis_name="c", subcore_axis_name="s",
                                    num_cores=2, num_subcores=_NTILE),
        scratch_types=[
            pltpu.VMEM((_CHUNK,), jnp.float32),
            pltpu.VMEM((_CHUNK,), jnp.float32),
            pltpu.VMEM((_B,), jnp.int32),
            pltpu.VMEM((128, _D), jnp.float32),
            pltpu.VMEM((128, _D), jnp.float32),
            pltpu.VMEM((128,), jnp.float32),
            pltpu.VMEM((128,), jnp.int32),
            pltpu.VMEM((128,), jnp.int32),
            pltpu.VMEM((32,), jnp.float32),
            pltpu.VMEM((32,), jnp.int32),
            pltpu.VMEM((16,), jnp.float32),
            pltpu.VMEM((16,), jnp.int32),
            pltpu.VMEM((16,), jnp.float32),
            pltpu.VMEM((16,), jnp.int32),
            pltpu.SMEM((1,), jnp.float32),
            pltpu.VMEM_SHARED((512,), jnp.float32),
            pltpu.VMEM_SHARED((512,), jnp.int32),
            pltpu.SemaphoreType.DMA,
            pltpu.SemaphoreType.DMA,
        ],
        compiler_params=pltpu.CompilerParams(needs_layout_passes=False),
        name="sc_sample",
    )(_sample_body)


def _log_sigmoid(x):
    return jnp.minimum(x, 0.0) - jnp.log(1.0 + jnp.exp(-jnp.abs(x)))


def _tgt_body(f_ref, wt_ref, bt_ref, out_ref):
    j = pl.program_id(0)
    lt = lax.dot_general(f_ref[...], wt_ref[...],
                         (((1,), (1,)), ((), ())),
                         preferred_element_type=jnp.float32) + bt_ref[0]
    s = jnp.sum(_log_sigmoid(lt))

    @pl.when(j == 0)
    def _init():
        out_ref[0, 0] = s

    @pl.when(j > 0)
    def _acc():
        out_ref[0, 0] += s


_tgt_kernel = pl.pallas_call(
    _tgt_body,
    grid=(_GRID,),
    in_specs=[
        pl.BlockSpec((_B, _D), lambda j: (0, 0)),
        pl.BlockSpec((_CT, _D), lambda j: (j, 0)),
        pl.BlockSpec((1, 1, _CT), lambda j: (j, 0, 0)),
    ],
    out_specs=pl.BlockSpec(memory_space=pltpu.SMEM),
    out_shape=jax.ShapeDtypeStruct((1, 1), jnp.float32),
    compiler_params=pltpu.CompilerParams(
        dimension_semantics=("arbitrary",),
    ),
)


def _noise_body(f_ref, wn_ref, bn_ref, part_ref, out_ref):
    zn = -(lax.dot_general(f_ref[...], wn_ref[...],
                           (((1,), (1,)), ((), ())),
                           preferred_element_type=jnp.float32)
           + bn_ref[0])
    lsn = _log_sigmoid(zn)
    msk = lax.broadcasted_iota(jnp.int32, (_B, 128), 1) < _NS
    s = jnp.sum(jnp.where(msk, lsn, 0.0)) + part_ref[0, 0]
    out_ref[0, 0] = -s / (_B * (_B + _NS))


_noise_kernel = pl.pallas_call(
    _noise_body,
    in_specs=[
        pl.BlockSpec((_B, _D), lambda: (0, 0)),
        pl.BlockSpec((128, _D), lambda: (0, 0)),
        pl.BlockSpec((1, 1, 128), lambda: (0, 0, 0)),
        pl.BlockSpec(memory_space=pltpu.SMEM),
    ],
    out_specs=pl.BlockSpec(memory_space=pltpu.SMEM),
    out_shape=jax.ShapeDtypeStruct((1, 1), jnp.float32),
)


def kernel(feature, targets, W, b, label_probs):
    e0 = jnp.exp(jax.random.gumbel(jax.random.key(42), (_V,), jnp.float32))
    lp_p = jnp.pad(label_probs.astype(jnp.float32), (0, _VP - _V))
    e0_p = jnp.pad(e0, (0, _VP - _V))
    b128 = jnp.pad(b.astype(jnp.float32), (0, _VP - _V)).reshape(_VP // 128, 128)
    tgt = targets.astype(jnp.int32)
    wt, bt = _make_gather_kernel()(tgt, W, b128)
    wn, bn = _make_sample_kernel()(lp_p, e0_p, tgt, W, b128)

    part = _tgt_kernel(feature, wt, bt.reshape(_GRID, 1, _CT))
    out = _noise_kernel(feature, wn, bn.reshape(1, 1, 128), part)
    return out[0, 0]
